# Initial kernel scaffold; baseline (speedup 1.0000x reference)
import jax
import jax.numpy as jnp
from jax.experimental import pallas as pl


def kernel(x, edge_index, edge_type, W_sl, b_sl, W1, b1, gamma, beta, W2, b2):
    raise NotImplementedError("write your pallas kernel here")


# speedup vs baseline: 6.6383x; 6.6383x over previous
import functools

import jax
import jax.numpy as jnp
from jax import lax
from jax.experimental import pallas as pl
from jax.experimental.pallas import tpu as pltpu
from jax.experimental.pallas import tpu_sc as plsc

N = 10000
E = 320000
D = 128
R = 4
BN_EPS = 1e-5

CH = 4
CW = 32
NT = 16
NC = 2
BE = 128
NB_E = 158
NHALF = 2
NBH = NB_E // NHALF
EPAD = NT * NB_E * BE
AGG_ROWS = R * N
AGG_PAD = NT * 2560

BN_BLK = 1000
NBLK = N // BN_BLK


def _sc_agg_body(gidx_hbm, sidx_hbm, xcm_hbm, out_hbm,
                 gidxb, sidxb, rows, agg, sem):
    c = lax.axis_index("c")
    s = lax.axis_index("s")
    zv = jnp.zeros((16,), jnp.float32)

    for p in range(CH // NC):
        k = c * (CH // NC) + p

        def zb_body(i, carry):
            rows[i, pl.ds(0, 16)] = zv
            rows[i, pl.ds(16, 16)] = zv
            return carry

        lax.fori_loop(0, BE, zb_body, 0)
        for q in range(2560 // BE):
            pltpu.sync_copy(rows, agg.at[pl.ds(s * 2560 + q * BE, BE)])
        plsc.subcore_barrier()

        for hf in range(NHALF):
            pltpu.sync_copy(gidx_hbm.at[k, s, hf], gidxb)
            pltpu.sync_copy(sidx_hbm.at[s, hf], sidxb)

            def body(j, carry):
                pltpu.async_copy(xcm_hbm.at[gidxb.at[j]], rows, sem).wait()
                pltpu.sync_copy(rows, agg.at[sidxb.at[j]], add=True)
                return carry

            lax.fori_loop(0, NBH, body, 0)
        plsc.subcore_barrier()

        pltpu.sync_copy(agg.at[pl.ds(s * 2560, 2560)],
                        out_hbm.at[k, pl.ds(s * 2560, 2560)])
        plsc.subcore_barrier()


@functools.cache
def _sc_agg():
    return pl.kernel(
        _sc_agg_body,
        mesh=plsc.VectorSubcoreMesh(core_axis_name="c", subcore_axis_name="s"),
        compiler_params=pltpu.CompilerParams(use_tc_tiling_on_sc=False),
        out_type=jax.ShapeDtypeStruct((CH, AGG_PAD, CW), jnp.float32),
        scratch_types=[
            pltpu.VMEM((NBH, BE), jnp.int32),
            pltpu.VMEM((NBH, BE), jnp.int32),
            pltpu.VMEM((BE, CW), jnp.float32),
            pltpu.VMEM_SHARED((AGG_PAD, CW), jnp.float32),
            pltpu.SemaphoreType.DMA,
        ],
    )


def _cat_agg(agg_ref):
    return jnp.concatenate(
        [agg_ref[0], agg_ref[1], agg_ref[2], agg_ref[3]], axis=-1)


def _stats_body(x_ref, agg_ref, w1_ref, b1_ref, out_ref):
    ib = pl.program_id(1)
    u = x_ref[...] + _cat_agg(agg_ref)
    h = jnp.dot(u, w1_ref[0], preferred_element_type=jnp.float32)
    h = h + b1_ref[0]
    st = jnp.stack([jnp.sum(h, axis=0), jnp.sum(h * h, axis=0)], axis=0)

    @pl.when(ib == 0)
    def _():
        out_ref[...] = st[None]

    @pl.when(ib != 0)
    def _():
        out_ref[...] += st[None]


def _mlp_body(x_ref, agg_ref, wsl_ref, bsl_ref, w1_ref, b1_ref, g_ref,
              be_ref, w2_ref, b2_ref, st_ref, out_ref):
    r = pl.program_id(1)
    xb = x_ref[...]
    u = xb + _cat_agg(agg_ref)
    h = jnp.dot(u, w1_ref[0], preferred_element_type=jnp.float32)
    h = h + b1_ref[0]
    mean = st_ref[0, 0] * (1.0 / N)
    var = st_ref[0, 1] * (1.0 / N) - mean * mean
    inv = lax.rsqrt(var + BN_EPS)
    hn = (h - mean[None, :]) * (inv * g_ref[0]) + be_ref[0]
    hr = jnp.maximum(hn, 0.0)
    contrib = jnp.dot(hr, w2_ref[0], preferred_element_type=jnp.float32)
    contrib = contrib + b2_ref[0]

    @pl.when(r == 0)
    def _():
        base = jnp.dot(xb, wsl_ref[...], preferred_element_type=jnp.float32)
        out_ref[...] = base + bsl_ref[...] + contrib

    @pl.when(r != 0)
    def _():
        out_ref[...] += contrib


def _tc_stats(x, aggc, W1, b1):
    return pl.pallas_call(
        _stats_body,
        grid=(R, NBLK),
        in_specs=[
            pl.BlockSpec((BN_BLK, D), lambda r, ib: (ib, 0)),
            pl.BlockSpec((CH, BN_BLK, CW), lambda r, ib: (0, r * NBLK + ib, 0)),
            pl.BlockSpec((1, D, D), lambda r, ib: (r, 0, 0)),
            pl.BlockSpec((1, 1, D), lambda r, ib: (r, 0, 0)),
        ],
        out_specs=pl.BlockSpec((1, 2, D), lambda r, ib: (r, 0, 0)),
        out_shape=jax.ShapeDtypeStruct((R, 2, D), jnp.float32),
    )(x, aggc, W1, b1[:, None, :])


def _tc_mlp(x, aggc, W_sl, b_sl2, W1, b1, gamma, beta, W2, b2, stats):
    return pl.pallas_call(
        _mlp_body,
        grid=(NBLK, R),
        in_specs=[
            pl.BlockSpec((BN_BLK, D), lambda ib, r: (ib, 0)),
            pl.BlockSpec((CH, BN_BLK, CW), lambda ib, r: (0, r * NBLK + ib, 0)),
            pl.BlockSpec((D, D), lambda ib, r: (0, 0)),
            pl.BlockSpec((1, D), lambda ib, r: (0, 0)),
            pl.BlockSpec((1, D, D), lambda ib, r: (r, 0, 0)),
            pl.BlockSpec((1, 1, D), lambda ib, r: (r, 0, 0)),
            pl.BlockSpec((1, 1, D), lambda ib, r: (r, 0, 0)),
            pl.BlockSpec((1, 1, D), lambda ib, r: (r, 0, 0)),
            pl.BlockSpec((1, D, D), lambda ib, r: (r, 0, 0)),
            pl.BlockSpec((1, 1, D), lambda ib, r: (r, 0, 0)),
            pl.BlockSpec((1, 2, D), lambda ib, r: (r, 0, 0)),
        ],
        out_specs=pl.BlockSpec((BN_BLK, D), lambda ib, r: (ib, 0)),
        out_shape=jax.ShapeDtypeStruct((N, D), jnp.float32),
    )(x, aggc, W_sl, b_sl2, W1, b1[:, None, :], gamma[:, None, :],
      beta[:, None, :], W2, b2[:, None, :], stats)


def kernel(x, edge_index, edge_type, W_sl, b_sl, W1, b1, gamma, beta, W2, b2):
    src = edge_index[0]
    dst = edge_index[1]
    pad = EPAD - E
    srcp = jnp.concatenate([src, jnp.zeros((pad,), jnp.int32)])
    sidx = jnp.concatenate(
        [edge_type * N + dst, jnp.full((pad,), AGG_ROWS, jnp.int32)])
    sidx_r = sidx.reshape(NT, NHALF, NBH, BE)
    offs = (jnp.arange(CH, dtype=jnp.int32) * N)[:, None, None, None, None]
    gidx_r = srcp.reshape(NT, NHALF, NBH, BE)[None] + offs
    xcm = x.reshape(N, CH, CW).transpose(1, 0, 2).reshape(CH * N, CW)

    aggc = _sc_agg()(gidx_r, sidx_r, xcm)
    stats = _tc_stats(x, aggc, W1, b1)
    return _tc_mlp(x, aggc, W_sl, b_sl[None, :], W1, b1, gamma, beta,
                   W2, b2, stats)


# speedup vs baseline: 7.4923x; 1.1287x over previous
import functools

import jax
import jax.numpy as jnp
from jax import lax
from jax.experimental import pallas as pl
from jax.experimental.pallas import tpu as pltpu
from jax.experimental.pallas import tpu_sc as plsc

N = 10000
E = 320000
D = 128
R = 4
BN_EPS = 1e-5

CH = 4
CW = 32
NT = 16
NC = 2
BE = 128
NB_E = 160
NHALF = 2
NBH = NB_E // NHALF
NBUF = 4
NGRP = NBH // NBUF
EPAD = NT * NB_E * BE
AGG_ROWS = R * N
AGG_PAD = NT * 2560

BN_BLK = 1000
NBLK = N // BN_BLK


def _sc_agg_body(gidx_hbm, sidx_hbm, xcm_hbm, out_hbm,
                 gidxb, sidxb, rows, agg, gsem, ssem):
    c = lax.axis_index("c")
    s = lax.axis_index("s")
    zv = jnp.zeros((16,), jnp.float32)

    def _drain_scatter(b, j):
        pltpu.make_async_copy(
            rows.at[b], agg.at[sidxb.at[j]], ssem.at[b]).wait()

    for p in range(CH // NC):
        k = c * (CH // NC) + p

        def zb_body(i, carry):
            rows[0, i, pl.ds(0, 16)] = zv
            rows[0, i, pl.ds(16, 16)] = zv
            return carry

        lax.fori_loop(0, BE, zb_body, 0)
        for q in range(2560 // BE):
            pltpu.sync_copy(rows.at[0], agg.at[pl.ds(s * 2560 + q * BE, BE)])
        plsc.subcore_barrier()

        for hf in range(NHALF):
            pltpu.sync_copy(gidx_hbm.at[k, s, hf], gidxb)
            pltpu.sync_copy(sidx_hbm.at[s, hf], sidxb)

            def group(g, carry):
                cps = []
                for b in range(NBUF):
                    j = g * NBUF + b

                    @pl.when(g > 0)
                    def _():
                        _drain_scatter(b, j)

                    cps.append(pltpu.async_copy(
                        xcm_hbm.at[gidxb.at[j]], rows.at[b], gsem.at[b]))
                for b in range(NBUF):
                    j = g * NBUF + b
                    cps[b].wait()
                    pltpu.async_copy(rows.at[b], agg.at[sidxb.at[j]],
                                     ssem.at[b], add=True)
                return carry

            lax.fori_loop(0, NGRP, group, 0)
            for b in range(NBUF):
                _drain_scatter(b, b)
        plsc.subcore_barrier()

        pltpu.sync_copy(agg.at[pl.ds(s * 2560, 2560)],
                        out_hbm.at[k, pl.ds(s * 2560, 2560)])
        plsc.subcore_barrier()


@functools.cache
def _sc_agg():
    return pl.kernel(
        _sc_agg_body,
        mesh=plsc.VectorSubcoreMesh(core_axis_name="c", subcore_axis_name="s"),
        compiler_params=pltpu.CompilerParams(use_tc_tiling_on_sc=False),
        out_type=jax.ShapeDtypeStruct((CH, AGG_PAD, CW), jnp.float32),
        scratch_types=[
            pltpu.VMEM((NBH, BE), jnp.int32),
            pltpu.VMEM((NBH, BE), jnp.int32),
            pltpu.VMEM((NBUF, BE, CW), jnp.float32),
            pltpu.VMEM_SHARED((AGG_PAD, CW), jnp.float32),
            pltpu.SemaphoreType.DMA((NBUF,)),
            pltpu.SemaphoreType.DMA((NBUF,)),
        ],
    )


def _cat_agg(agg_ref):
    return jnp.concatenate(
        [agg_ref[0], agg_ref[1], agg_ref[2], agg_ref[3]], axis=-1)


def _stats_body(x_ref, agg_ref, w1_ref, b1_ref, out_ref):
    ib = pl.program_id(1)
    u = x_ref[...] + _cat_agg(agg_ref)
    h = jnp.dot(u, w1_ref[0], preferred_element_type=jnp.float32)
    h = h + b1_ref[0]
    st = jnp.stack([jnp.sum(h, axis=0), jnp.sum(h * h, axis=0)], axis=0)

    @pl.when(ib == 0)
    def _():
        out_ref[...] = st[None]

    @pl.when(ib != 0)
    def _():
        out_ref[...] += st[None]


def _mlp_body(x_ref, agg_ref, wsl_ref, bsl_ref, w1_ref, b1_ref, g_ref,
              be_ref, w2_ref, b2_ref, st_ref, out_ref):
    r = pl.program_id(1)
    xb = x_ref[...]
    u = xb + _cat_agg(agg_ref)
    h = jnp.dot(u, w1_ref[0], preferred_element_type=jnp.float32)
    h = h + b1_ref[0]
    mean = st_ref[0, 0] * (1.0 / N)
    var = st_ref[0, 1] * (1.0 / N) - mean * mean
    inv = lax.rsqrt(var + BN_EPS)
    hn = (h - mean[None, :]) * (inv * g_ref[0]) + be_ref[0]
    hr = jnp.maximum(hn, 0.0)
    contrib = jnp.dot(hr, w2_ref[0], preferred_element_type=jnp.float32)
    contrib = contrib + b2_ref[0]

    @pl.when(r == 0)
    def _():
        base = jnp.dot(xb, wsl_ref[...], preferred_element_type=jnp.float32)
        out_ref[...] = base + bsl_ref[...] + contrib

    @pl.when(r != 0)
    def _():
        out_ref[...] += contrib


def _tc_stats(x, aggc, W1, b1):
    return pl.pallas_call(
        _stats_body,
        grid=(R, NBLK),
        in_specs=[
            pl.BlockSpec((BN_BLK, D), lambda r, ib: (ib, 0)),
            pl.BlockSpec((CH, BN_BLK, CW), lambda r, ib: (0, r * NBLK + ib, 0)),
            pl.BlockSpec((1, D, D), lambda r, ib: (r, 0, 0)),
            pl.BlockSpec((1, 1, D), lambda r, ib: (r, 0, 0)),
        ],
        out_specs=pl.BlockSpec((1, 2, D), lambda r, ib: (r, 0, 0)),
        out_shape=jax.ShapeDtypeStruct((R, 2, D), jnp.float32),
    )(x, aggc, W1, b1[:, None, :])


def _tc_mlp(x, aggc, W_sl, b_sl2, W1, b1, gamma, beta, W2, b2, stats):
    return pl.pallas_call(
        _mlp_body,
        grid=(NBLK, R),
        in_specs=[
            pl.BlockSpec((BN_BLK, D), lambda ib, r: (ib, 0)),
            pl.BlockSpec((CH, BN_BLK, CW), lambda ib, r: (0, r * NBLK + ib, 0)),
            pl.BlockSpec((D, D), lambda ib, r: (0, 0)),
            pl.BlockSpec((1, D), lambda ib, r: (0, 0)),
            pl.BlockSpec((1, D, D), lambda ib, r: (r, 0, 0)),
            pl.BlockSpec((1, 1, D), lambda ib, r: (r, 0, 0)),
            pl.BlockSpec((1, 1, D), lambda ib, r: (r, 0, 0)),
            pl.BlockSpec((1, 1, D), lambda ib, r: (r, 0, 0)),
            pl.BlockSpec((1, D, D), lambda ib, r: (r, 0, 0)),
            pl.BlockSpec((1, 1, D), lambda ib, r: (r, 0, 0)),
            pl.BlockSpec((1, 2, D), lambda ib, r: (r, 0, 0)),
        ],
        out_specs=pl.BlockSpec((BN_BLK, D), lambda ib, r: (ib, 0)),
        out_shape=jax.ShapeDtypeStruct((N, D), jnp.float32),
    )(x, aggc, W_sl, b_sl2, W1, b1[:, None, :], gamma[:, None, :],
      beta[:, None, :], W2, b2[:, None, :], stats)


def kernel(x, edge_index, edge_type, W_sl, b_sl, W1, b1, gamma, beta, W2, b2):
    src = edge_index[0]
    dst = edge_index[1]
    pad = EPAD - E
    srcp = jnp.concatenate([src, jnp.zeros((pad,), jnp.int32)])
    sidx = jnp.concatenate(
        [edge_type * N + dst, jnp.full((pad,), AGG_ROWS, jnp.int32)])
    sidx_r = sidx.reshape(NT, NHALF, NBH, BE)
    offs = (jnp.arange(CH, dtype=jnp.int32) * N)[:, None, None, None, None]
    gidx_r = srcp.reshape(NT, NHALF, NBH, BE)[None] + offs
    xcm = x.reshape(N, CH, CW).transpose(1, 0, 2).reshape(CH * N, CW)

    aggc = _sc_agg()(gidx_r, sidx_r, xcm)
    stats = _tc_stats(x, aggc, W1, b1)
    return _tc_mlp(x, aggc, W_sl, b_sl[None, :], W1, b1, gamma, beta,
                   W2, b2, stats)


# speedup vs baseline: 8.7655x; 1.1699x over previous
import functools

import jax
import jax.numpy as jnp
from jax import lax
from jax.experimental import pallas as pl
from jax.experimental.pallas import tpu as pltpu
from jax.experimental.pallas import tpu_sc as plsc

N = 10000
E = 320000
D = 128
R = 4
BN_EPS = 1e-5

CH = 4
CW = 32
NT = 16
NC = 2
BE = 128
NB_E = 160
NHALF = 2
NBH = NB_E // NHALF
NBUF = 4
NGRP = NBH // NBUF
EPAD = NT * NB_E * BE
AGG_ROWS = R * N
AGG_PAD = NT * 2560

BN_BLK = 1000
NBLK = N // BN_BLK


def _sc_agg_body(src_hbm, sidx_hbm, xcm_hbm, out_hbm,
                 srcb, sidxb, rows, agg, gsem, ssem):
    c = lax.axis_index("c")
    s = lax.axis_index("s")
    zv = jnp.zeros((16,), jnp.float32)

    def _drain_scatter(b, j):
        pltpu.make_async_copy(
            rows.at[b], agg.at[sidxb.at[j]], ssem.at[b]).wait()

    for p in range(CH // NC):
        k = c * (CH // NC) + p
        koff = k * N

        def zb_body(i, carry):
            rows[0, i, pl.ds(0, 16)] = zv
            rows[0, i, pl.ds(16, 16)] = zv
            return carry

        lax.fori_loop(0, BE, zb_body, 0)
        for q in range(2560 // BE):
            pltpu.sync_copy(rows.at[0], agg.at[pl.ds(s * 2560 + q * BE, BE)])
        plsc.subcore_barrier()

        for hf in range(NHALF):
            pltpu.sync_copy(src_hbm.at[s, hf], srcb)
            pltpu.sync_copy(sidx_hbm.at[s, hf], sidxb)

            def off_body(j, carry):
                for t in range(BE // 16):
                    sl = pl.ds(t * 16, 16)
                    srcb[j, sl] = srcb[j, sl] + koff
                return carry

            lax.fori_loop(0, NBH, off_body, 0)

            def group(g, carry):
                cps = []
                for b in range(NBUF):
                    j = g * NBUF + b

                    @pl.when(g > 0)
                    def _():
                        _drain_scatter(b, j)

                    cps.append(pltpu.async_copy(
                        xcm_hbm.at[srcb.at[j]], rows.at[b], gsem.at[b]))
                for b in range(NBUF):
                    j = g * NBUF + b
                    cps[b].wait()
                    pltpu.async_copy(rows.at[b], agg.at[sidxb.at[j]],
                                     ssem.at[b], add=True)
                return carry

            lax.fori_loop(0, NGRP, group, 0)
            for b in range(NBUF):
                _drain_scatter(b, b)
        plsc.subcore_barrier()

        pltpu.sync_copy(agg.at[pl.ds(s * 2560, 2560)],
                        out_hbm.at[pl.ds(s * 2560, 2560), pl.ds(k * CW, CW)])
        plsc.subcore_barrier()


@functools.cache
def _sc_agg():
    return pl.kernel(
        _sc_agg_body,
        mesh=plsc.VectorSubcoreMesh(core_axis_name="c", subcore_axis_name="s"),
        compiler_params=pltpu.CompilerParams(use_tc_tiling_on_sc=False),
        out_type=jax.ShapeDtypeStruct((AGG_PAD, D), jnp.float32),
        scratch_types=[
            pltpu.VMEM((NBH, BE), jnp.int32),
            pltpu.VMEM((NBH, BE), jnp.int32),
            pltpu.VMEM((NBUF, BE, CW), jnp.float32),
            pltpu.VMEM_SHARED((AGG_PAD, CW), jnp.float32),
            pltpu.SemaphoreType.DMA((NBUF,)),
            pltpu.SemaphoreType.DMA((NBUF,)),
        ],
    )


def _stats_body(x_ref, agg_ref, w1_ref, b1_ref, out_ref):
    ib = pl.program_id(1)
    u = x_ref[...] + agg_ref[...]
    h = jnp.dot(u, w1_ref[0], preferred_element_type=jnp.float32)
    h = h + b1_ref[0]
    st = jnp.stack([jnp.sum(h, axis=0), jnp.sum(h * h, axis=0)], axis=0)

    @pl.when(ib == 0)
    def _():
        out_ref[...] = st[None]

    @pl.when(ib != 0)
    def _():
        out_ref[...] += st[None]


def _mlp_body(x_ref, agg_ref, wsl_ref, bsl_ref, w1_ref, b1_ref, g_ref,
              be_ref, w2_ref, b2_ref, st_ref, out_ref):
    r = pl.program_id(1)
    xb = x_ref[...]
    u = xb + agg_ref[...]
    h = jnp.dot(u, w1_ref[0], preferred_element_type=jnp.float32)
    h = h + b1_ref[0]
    mean = st_ref[0, 0] * (1.0 / N)
    var = st_ref[0, 1] * (1.0 / N) - mean * mean
    inv = lax.rsqrt(var + BN_EPS)
    hn = (h - mean[None, :]) * (inv * g_ref[0]) + be_ref[0]
    hr = jnp.maximum(hn, 0.0)
    contrib = jnp.dot(hr, w2_ref[0], preferred_element_type=jnp.float32)
    contrib = contrib + b2_ref[0]

    @pl.when(r == 0)
    def _():
        base = jnp.dot(xb, wsl_ref[...], preferred_element_type=jnp.float32)
        out_ref[...] = base + bsl_ref[...] + contrib

    @pl.when(r != 0)
    def _():
        out_ref[...] += contrib


def _tc_stats(x, aggc, W1, b1):
    return pl.pallas_call(
        _stats_body,
        grid=(R, NBLK),
        in_specs=[
            pl.BlockSpec((BN_BLK, D), lambda r, ib: (ib, 0)),
            pl.BlockSpec((BN_BLK, D), lambda r, ib: (r * NBLK + ib, 0)),
            pl.BlockSpec((1, D, D), lambda r, ib: (r, 0, 0)),
            pl.BlockSpec((1, 1, D), lambda r, ib: (r, 0, 0)),
        ],
        out_specs=pl.BlockSpec((1, 2, D), lambda r, ib: (r, 0, 0)),
        out_shape=jax.ShapeDtypeStruct((R, 2, D), jnp.float32),
    )(x, aggc, W1, b1[:, None, :])


def _tc_mlp(x, aggc, W_sl, b_sl2, W1, b1, gamma, beta, W2, b2, stats):
    return pl.pallas_call(
        _mlp_body,
        grid=(NBLK, R),
        in_specs=[
            pl.BlockSpec((BN_BLK, D), lambda ib, r: (ib, 0)),
            pl.BlockSpec((BN_BLK, D), lambda ib, r: (r * NBLK + ib, 0)),
            pl.BlockSpec((D, D), lambda ib, r: (0, 0)),
            pl.BlockSpec((1, D), lambda ib, r: (0, 0)),
            pl.BlockSpec((1, D, D), lambda ib, r: (r, 0, 0)),
            pl.BlockSpec((1, 1, D), lambda ib, r: (r, 0, 0)),
            pl.BlockSpec((1, 1, D), lambda ib, r: (r, 0, 0)),
            pl.BlockSpec((1, 1, D), lambda ib, r: (r, 0, 0)),
            pl.BlockSpec((1, D, D), lambda ib, r: (r, 0, 0)),
            pl.BlockSpec((1, 1, D), lambda ib, r: (r, 0, 0)),
            pl.BlockSpec((1, 2, D), lambda ib, r: (r, 0, 0)),
        ],
        out_specs=pl.BlockSpec((BN_BLK, D), lambda ib, r: (ib, 0)),
        out_shape=jax.ShapeDtypeStruct((N, D), jnp.float32),
    )(x, aggc, W_sl, b_sl2, W1, b1[:, None, :], gamma[:, None, :],
      beta[:, None, :], W2, b2[:, None, :], stats)


def kernel(x, edge_index, edge_type, W_sl, b_sl, W1, b1, gamma, beta, W2, b2):
    src = edge_index[0]
    dst = edge_index[1]
    pad = EPAD - E
    srcp = jnp.concatenate([src, jnp.zeros((pad,), jnp.int32)])
    sidx = jnp.concatenate(
        [edge_type * N + dst, jnp.full((pad,), AGG_ROWS, jnp.int32)])
    sidx_r = sidx.reshape(NT, NHALF, NBH, BE)
    src_r = srcp.reshape(NT, NHALF, NBH, BE)
    xcm = x.reshape(N, CH, CW).transpose(1, 0, 2).reshape(CH * N, CW)

    aggc = _sc_agg()(src_r, sidx_r, xcm)
    stats = _tc_stats(x, aggc, W1, b1)
    return _tc_mlp(x, aggc, W_sl, b_sl[None, :], W1, b1, gamma, beta,
                   W2, b2, stats)
